# baseline (speedup 1.0000x reference)
import functools

import jax
import jax.numpy as jnp
from jax import lax
from jax.experimental import pallas as pl
from jax.experimental.pallas import tpu as pltpu
from jax.experimental.pallas import tpu_sc as plsc

B = 16384
D = 32
PK = 8
DW = 128
NC = 2
NS = 16
NW = NC * NS
BPW = B // NW
CH = 128
NCH = BPW // CH

QU = 128000
QI = 12800
PROWS_U = 5120
PROWS_I = 6400

BLK = 8192


def _pack_body(*refs):
    xs = refs[:PK]
    o = refs[PK]
    X = jnp.concatenate([x[...] for x in xs], axis=0)
    n = PK * D
    eye = (lax.broadcasted_iota(jnp.int32, (n, n), 0)
           == lax.broadcasted_iota(jnp.int32, (n, n), 1)).astype(jnp.float32)
    y = lax.dot_general(X, eye, (((0,), (0,)), ((), ())),
                        preferred_element_type=jnp.float32)
    lo_bits = lax.bitcast_convert_type(y[:, 0:DW], jnp.uint32)
    hi_bits = lax.bitcast_convert_type(y[:, DW:2 * DW], jnp.uint32)
    lo16 = (lo_bits + jnp.uint32(0x8000)) >> 16
    hi16 = (hi_bits + jnp.uint32(0x8000)) & jnp.uint32(0xFFFF0000)
    o[...] = lax.bitcast_convert_type(hi16 | lo16, jnp.int32)


def _pack(tT, q, prows):
    steps = q // prows
    maxb = (tT.shape[1] + prows - 1) // prows - 1
    in_specs = [
        pl.BlockSpec((D, prows),
                     lambda g, k=k: (0, jnp.minimum(k * steps + g, maxb)))
        for k in range(PK)
    ]
    return pl.pallas_call(
        _pack_body,
        grid=(steps,),
        in_specs=in_specs,
        out_specs=pl.BlockSpec((prows, DW), lambda g: (g, 0)),
        out_shape=jax.ShapeDtypeStruct((q, DW), jnp.int32),
    )(*([tT] * PK))


def _gather_body(tab, ids, out, idx, rows, sem):
    wid = lax.axis_index("s") * NC + lax.axis_index("c")
    base = wid * BPW
    pltpu.sync_copy(ids.at[wid], idx)
    copies = [
        pltpu.async_copy(tab.at[idx.at[j]], rows.at[j], sem)
        for j in range(NCH)
    ]
    for c in copies:
        c.wait()
    for j in range(NCH):
        pltpu.sync_copy(rows.at[j], out.at[pl.ds(base + j * CH, CH)])


def _sc_gather(tab, id3):
    mesh = plsc.VectorSubcoreMesh(core_axis_name="c", subcore_axis_name="s")
    fn = functools.partial(
        pl.kernel,
        mesh=mesh,
        out_type=jax.ShapeDtypeStruct((B, DW), jnp.int32),
        scratch_types=[
            pltpu.VMEM((NCH, CH), jnp.int32),
            pltpu.VMEM((NCH, CH, DW), jnp.int32),
            pltpu.SemaphoreType.DMA,
        ],
    )(_gather_body)
    return fn(tab, id3)


def _mlp_body(u, i, ulo, ilo, w1, b1, w2, b2, w3t, b3, o):
    lgrp = lax.broadcasted_iota(jnp.int32, (BLK, DW), 1) // D
    pick = (lax.broadcasted_iota(jnp.int32, (DW, D), 0) % D
            == lax.broadcasted_iota(jnp.int32, (DW, D), 1)).astype(jnp.float32)

    def select(raw, lo):
        k = lo[...].reshape(BLK, 1)
        bits = lax.bitcast_convert_type(raw[...], jnp.uint32)
        lowf = lax.bitcast_convert_type(bits << 16, jnp.float32)
        highf = lax.bitcast_convert_type(
            bits & jnp.uint32(0xFFFF0000), jnp.float32)
        chosen = jnp.where(k < 4, lowf, highf)
        masked = jnp.where(lgrp == k % 4, chosen, 0.0)
        return jnp.dot(masked, pick, preferred_element_type=jnp.float32)

    x = jnp.concatenate([select(u, ulo), select(i, ilo)], axis=1)
    h = jnp.maximum(
        jnp.dot(x, w1[...], preferred_element_type=jnp.float32) + b1[...], 0.0)
    h = jnp.maximum(
        jnp.dot(h, w2[...], preferred_element_type=jnp.float32) + b2[...], 0.0)
    z = jnp.sum(h * w3t[...], axis=1) + b3[0, 0]
    o[...] = jax.nn.sigmoid(z)


def _tc_mlp(u_raw, i_raw, u_lo, i_lo, W1, b1, W2, b2, W3, b3):
    b1r = b1.reshape(1, -1)
    b2r = b2.reshape(1, -1)
    w3t = W3.reshape(1, -1)
    b3r = b3.reshape(1, 1)
    grid = (B // BLK,)
    return pl.pallas_call(
        _mlp_body,
        grid=grid,
        in_specs=[
            pl.BlockSpec((BLK, DW), lambda g: (g, 0)),
            pl.BlockSpec((BLK, DW), lambda g: (g, 0)),
            pl.BlockSpec((BLK,), lambda g: (g,)),
            pl.BlockSpec((BLK,), lambda g: (g,)),
            pl.BlockSpec(W1.shape, lambda g: (0, 0)),
            pl.BlockSpec(b1r.shape, lambda g: (0, 0)),
            pl.BlockSpec(W2.shape, lambda g: (0, 0)),
            pl.BlockSpec(b2r.shape, lambda g: (0, 0)),
            pl.BlockSpec(w3t.shape, lambda g: (0, 0)),
            pl.BlockSpec(memory_space=pltpu.SMEM),
        ],
        out_specs=pl.BlockSpec((BLK,), lambda g: (g,)),
        out_shape=jax.ShapeDtypeStruct((B,), jnp.float32),
    )(u_raw, i_raw, u_lo, i_lo, W1, b1r, W2, b2r, w3t, b3r)


def kernel(user_table, item_table, W1, b1, W2, b2, W3, b3, user_ids, item_ids):
    uids = user_ids.astype(jnp.int32)
    iids = item_ids.astype(jnp.int32)
    uid3 = (uids % QU).reshape(NW, NCH, CH)
    iid3 = (iids % QI).reshape(NW, NCH, CH)
    u_lo = uids // QU
    i_lo = iids // QI
    it4 = _pack(item_table.T, QI, PROWS_I)
    i_raw = _sc_gather(it4, iid3)
    ut4 = _pack(user_table.T, QU, PROWS_U)
    u_raw = _sc_gather(ut4, uid3)
    return _tc_mlp(u_raw, i_raw, u_lo, i_lo, W1, b1, W2, b2, W3, b3)


# speedup vs baseline: 1.0120x; 1.0120x over previous
import functools

import jax
import jax.numpy as jnp
from jax import lax
from jax.experimental import pallas as pl
from jax.experimental.pallas import tpu as pltpu
from jax.experimental.pallas import tpu_sc as plsc

B = 16384
D = 32
PK = 8
DW = 128
NC = 2
NS = 16
NW = NC * NS
BPW = B // NW
CH = 128
NCH = BPW // CH

QU = 128000
QI = 12800
PROWS_U = 5120
PROWS_I = 6400

BLK = 4096


def _pack_body(*refs):
    xs = refs[:PK]
    o = refs[PK]
    X = jnp.concatenate([x[...] for x in xs], axis=0)
    n = PK * D
    eye = (lax.broadcasted_iota(jnp.int32, (n, n), 0)
           == lax.broadcasted_iota(jnp.int32, (n, n), 1)).astype(jnp.float32)
    y = lax.dot_general(X, eye, (((0,), (0,)), ((), ())),
                        preferred_element_type=jnp.float32)
    lo_bits = lax.bitcast_convert_type(y[:, 0:DW], jnp.uint32)
    hi_bits = lax.bitcast_convert_type(y[:, DW:2 * DW], jnp.uint32)
    lo16 = (lo_bits + jnp.uint32(0x8000)) >> 16
    hi16 = (hi_bits + jnp.uint32(0x8000)) & jnp.uint32(0xFFFF0000)
    o[...] = lax.bitcast_convert_type(hi16 | lo16, jnp.int32)


def _pack(tT, q, prows):
    steps = q // prows
    maxb = (tT.shape[1] + prows - 1) // prows - 1
    in_specs = [
        pl.BlockSpec((D, prows),
                     lambda g, k=k: (0, jnp.minimum(k * steps + g, maxb)))
        for k in range(PK)
    ]
    return pl.pallas_call(
        _pack_body,
        grid=(steps,),
        in_specs=in_specs,
        out_specs=pl.BlockSpec((prows, DW), lambda g: (g, 0)),
        out_shape=jax.ShapeDtypeStruct((q, DW), jnp.int32),
    )(*([tT] * PK))


def _gather_body(tab, ids, out, idx, rows, sem):
    wid = lax.axis_index("s") * NC + lax.axis_index("c")
    base = wid * BPW
    pltpu.sync_copy(ids.at[wid], idx)
    copies = [
        pltpu.async_copy(tab.at[idx.at[j]], rows.at[j], sem)
        for j in range(NCH)
    ]
    for c in copies:
        c.wait()
    for j in range(NCH):
        pltpu.sync_copy(rows.at[j], out.at[pl.ds(base + j * CH, CH)])


def _sc_gather(tab, id3):
    mesh = plsc.VectorSubcoreMesh(core_axis_name="c", subcore_axis_name="s")
    fn = functools.partial(
        pl.kernel,
        mesh=mesh,
        out_type=jax.ShapeDtypeStruct((B, DW), jnp.int32),
        scratch_types=[
            pltpu.VMEM((NCH, CH), jnp.int32),
            pltpu.VMEM((NCH, CH, DW), jnp.int32),
            pltpu.SemaphoreType.DMA,
        ],
    )(_gather_body)
    return fn(tab, id3)


def _mlp_body(u, i, ulo, ilo, w1, b1, w2, b2, w3t, b3, o):
    lgrp = lax.broadcasted_iota(jnp.int32, (BLK, DW), 1) // D
    pick = (lax.broadcasted_iota(jnp.int32, (DW, D), 0) % D
            == lax.broadcasted_iota(jnp.int32, (DW, D), 1)).astype(jnp.float32)

    def select(raw, lo):
        k = lo[...].reshape(BLK, 1)
        shift = jnp.where(k < 4, jnp.uint32(16), jnp.uint32(0))
        bits = lax.bitcast_convert_type(raw[...], jnp.uint32)
        chosen = lax.bitcast_convert_type(
            (bits << shift) & jnp.uint32(0xFFFF0000), jnp.float32)
        masked = jnp.where(lgrp == k % 4, chosen, 0.0)
        return jnp.dot(masked, pick, preferred_element_type=jnp.float32)

    x = jnp.concatenate([select(u, ulo), select(i, ilo)], axis=1)
    h = jnp.maximum(
        jnp.dot(x, w1[...], preferred_element_type=jnp.float32) + b1[...], 0.0)
    h = jnp.maximum(
        jnp.dot(h, w2[...], preferred_element_type=jnp.float32) + b2[...], 0.0)
    z = jnp.sum(h * w3t[...], axis=1) + b3[0, 0]
    o[...] = jax.nn.sigmoid(z)


def _tc_mlp(u_raw, i_raw, u_lo, i_lo, W1, b1, W2, b2, W3, b3):
    b1r = b1.reshape(1, -1)
    b2r = b2.reshape(1, -1)
    w3t = W3.reshape(1, -1)
    b3r = b3.reshape(1, 1)
    grid = (B // BLK,)
    return pl.pallas_call(
        _mlp_body,
        grid=grid,
        in_specs=[
            pl.BlockSpec((BLK, DW), lambda g: (g, 0)),
            pl.BlockSpec((BLK, DW), lambda g: (g, 0)),
            pl.BlockSpec((BLK,), lambda g: (g,)),
            pl.BlockSpec((BLK,), lambda g: (g,)),
            pl.BlockSpec(W1.shape, lambda g: (0, 0)),
            pl.BlockSpec(b1r.shape, lambda g: (0, 0)),
            pl.BlockSpec(W2.shape, lambda g: (0, 0)),
            pl.BlockSpec(b2r.shape, lambda g: (0, 0)),
            pl.BlockSpec(w3t.shape, lambda g: (0, 0)),
            pl.BlockSpec(memory_space=pltpu.SMEM),
        ],
        out_specs=pl.BlockSpec((BLK,), lambda g: (g,)),
        out_shape=jax.ShapeDtypeStruct((B,), jnp.float32),
    )(u_raw, i_raw, u_lo, i_lo, W1, b1r, W2, b2r, w3t, b3r)


def kernel(user_table, item_table, W1, b1, W2, b2, W3, b3, user_ids, item_ids):
    uids = user_ids.astype(jnp.int32)
    iids = item_ids.astype(jnp.int32)
    uid3 = (uids % QU).reshape(NW, NCH, CH)
    iid3 = (iids % QI).reshape(NW, NCH, CH)
    u_lo = uids // QU
    i_lo = iids // QI
    it4 = _pack(item_table.T, QI, PROWS_I)
    i_raw = _sc_gather(it4, iid3)
    ut4 = _pack(user_table.T, QU, PROWS_U)
    u_raw = _sc_gather(ut4, uid3)
    return _tc_mlp(u_raw, i_raw, u_lo, i_lo, W1, b1, W2, b2, W3, b3)
